# baseline (speedup 1.0000x reference)
import functools

import jax
import jax.numpy as jnp
from jax import lax
from jax.experimental import pallas as pl
from jax.experimental.pallas import tpu as pltpu
from jax.experimental.pallas import tpu_sc as plsc

_ZERO_WEIGHTING = 0.5
_NONZERO_WEIGHTING = 1.0

_NC, _NS, _L = 2, 16, 16
_NW = _NC * _NS
_PLANES, _H, _W = 384, 224, 224
_SC_PLANES = 64
_TC_PLANES = _PLANES - _SC_PLANES
_TC_BLOCK = 16
_PPW = _SC_PLANES // _NW
_HH = _H // 2
_CHUNKS = _PPW * 2
_GROUPS = _W // _L


def _sc_body(rec_hbm, tgt_hbm, out_hbm, rec_v, tgt_v, acc_v, sem_r, sem_t):
    wid = lax.axis_index("s") * _NC + lax.axis_index("c")
    base = wid * _PPW

    def rec_copy(c, buf):
        plane = base + c // 2
        r0 = (c % 2) * _HH
        return pltpu.make_async_copy(
            rec_hbm.at[plane, pl.ds(r0, _HH), :], rec_v.at[buf], sem_r.at[buf]
        )

    def tgt_copy(c, buf):
        plane = base + c // 2
        r0 = (c % 2) * _HH
        return pltpu.make_async_copy(
            tgt_hbm.at[plane, pl.ds(r0, _HH), :], tgt_v.at[buf], sem_t.at[buf]
        )

    def start(c, buf):
        rec_copy(c, buf).start()
        tgt_copy(c, buf).start()

    def wait(c, buf):
        rec_copy(c, buf).wait()
        tgt_copy(c, buf).wait()

    def compute(buf, carry):
        def row_body(row, cr):
            ssq, sab, cnt = cr
            for g in range(_GROUPS):
                rr = rec_v[buf, row, pl.ds(g * _L, _L)]
                tt = tgt_v[buf, row, pl.ds(g * _L, _L)]
                z = tt == 0.0
                d = rr - tt
                ssq = ssq + jnp.where(z, d * d, 0.0)
                sab = sab + jnp.where(z, 0.0, jnp.abs(d))
                cnt = cnt + jnp.where(z, 1.0, 0.0)
            return ssq, sab, cnt

        return lax.fori_loop(0, _HH, row_body, carry)

    start(0, 0)
    start(1, 1)
    zeros = jnp.zeros((_L,), jnp.float32)
    carry0 = (zeros, zeros, zeros)

    def pair_body(k, carry):
        c0 = 2 * k
        wait(c0, 0)

        @pl.when(c0 + 2 < _CHUNKS)
        def _():
            start(c0 + 2, 0)

        carry = compute(0, carry)
        c1 = c0 + 1
        wait(c1, 1)

        @pl.when(c1 + 2 < _CHUNKS)
        def _():
            start(c1 + 2, 1)

        carry = compute(1, carry)
        return carry

    ssq, sab, cnt = lax.fori_loop(0, _CHUNKS // 2, pair_body, carry0)
    acc_v[0, :] = ssq
    acc_v[1, :] = sab
    acc_v[2, :] = cnt
    pltpu.sync_copy(acc_v, out_hbm.at[wid])


_sc_loss = functools.partial(
    pl.kernel,
    out_type=jax.ShapeDtypeStruct((_NW, 3, _L), jnp.float32),
    mesh=plsc.VectorSubcoreMesh(
        core_axis_name="c", subcore_axis_name="s", num_cores=_NC, num_subcores=_NS
    ),
    scratch_types=[
        pltpu.VMEM((2, _HH, _W), jnp.float32),
        pltpu.VMEM((2, _HH, _W), jnp.float32),
        pltpu.VMEM((3, _L), jnp.float32),
        pltpu.SemaphoreType.DMA((2,)),
        pltpu.SemaphoreType.DMA((2,)),
    ],
)(_sc_body)


def _tc_body(rec_ref, tgt_ref, out_ref, acc_ref):
    i = pl.program_id(0)
    n = pl.num_programs(0)

    t = tgt_ref[...]
    r = rec_ref[...]
    zero = t == 0.0
    d = r - t
    ssq = jnp.sum(jnp.where(zero, d * d, 0.0))
    sab = jnp.sum(jnp.where(zero, 0.0, jnp.abs(d)))
    nz = jnp.sum(zero.astype(jnp.float32))

    @pl.when(i == 0)
    def _init():
        acc_ref[0] = 0.0
        acc_ref[1] = 0.0
        acc_ref[2] = 0.0

    acc_ref[0] += ssq
    acc_ref[1] += sab
    acc_ref[2] += nz

    @pl.when(i == n - 1)
    def _finish():
        out_ref[0] = acc_ref[0]
        out_ref[1] = acc_ref[1]
        out_ref[2] = acc_ref[2]


def _tc_partials(rec, tgt):
    grid = _TC_PLANES // _TC_BLOCK
    off = _SC_PLANES // _TC_BLOCK
    return pl.pallas_call(
        _tc_body,
        grid=(grid,),
        in_specs=[
            pl.BlockSpec((_TC_BLOCK, _H, _W), lambda i: (i + off, 0, 0)),
            pl.BlockSpec((_TC_BLOCK, _H, _W), lambda i: (i + off, 0, 0)),
        ],
        out_specs=pl.BlockSpec(memory_space=pltpu.SMEM),
        out_shape=jax.ShapeDtypeStruct((3,), jnp.float32),
        scratch_shapes=[pltpu.SMEM((3,), jnp.float32)],
    )(rec, tgt)


def kernel(reconstructed_image, target_image):
    total_n = float(reconstructed_image.size)
    rec = reconstructed_image.reshape(_PLANES, _H, _W)
    tgt = target_image.reshape(_PLANES, _H, _W)

    sc_partials = _sc_loss(rec, tgt)
    tc_partials = _tc_partials(rec, tgt)

    ssq = jnp.sum(sc_partials[:, 0, :]) + tc_partials[0]
    sab = jnp.sum(sc_partials[:, 1, :]) + tc_partials[1]
    n_zero = jnp.sum(sc_partials[:, 2, :]) + tc_partials[2]
    n_nonzero = total_n - n_zero
    zero_loss = jnp.where(n_zero > 0, ssq / jnp.maximum(n_zero, 1.0), 0.0)
    nonzero_loss = jnp.where(n_nonzero > 0, sab / jnp.maximum(n_nonzero, 1.0), 0.0)
    return _ZERO_WEIGHTING * zero_loss + _NONZERO_WEIGHTING * nonzero_loss


# speedup vs baseline: 1.2121x; 1.2121x over previous
import functools

import jax
import jax.numpy as jnp
from jax.experimental import pallas as pl
from jax.experimental.pallas import tpu as pltpu

_ZERO_WEIGHTING = 0.5
_NONZERO_WEIGHTING = 1.0

_PLANES = 384
_H = 224
_W = 224
_BLOCK_PLANES = 12


def _loss_body(rec_ref, tgt_ref, out_ref, acc_ref, *, total_n):
    i = pl.program_id(0)
    n = pl.num_programs(0)

    t = tgt_ref[...]
    r = rec_ref[...]
    zero = t == 0.0
    d = r - t
    ssq = jnp.sum(jnp.where(zero, d * d, 0.0))
    sab = jnp.sum(jnp.where(zero, 0.0, jnp.abs(d)))
    nz = jnp.sum(zero.astype(jnp.float32))

    @pl.when(i == 0)
    def _init():
        acc_ref[0] = 0.0
        acc_ref[1] = 0.0
        acc_ref[2] = 0.0

    acc_ref[0] += ssq
    acc_ref[1] += sab
    acc_ref[2] += nz

    @pl.when(i == n - 1)
    def _finish():
        n_zero = acc_ref[2]
        n_nonzero = total_n - n_zero
        zero_loss = jnp.where(n_zero > 0, acc_ref[0] / jnp.maximum(n_zero, 1.0), 0.0)
        nonzero_loss = jnp.where(
            n_nonzero > 0, acc_ref[1] / jnp.maximum(n_nonzero, 1.0), 0.0
        )
        out_ref[0] = _ZERO_WEIGHTING * zero_loss + _NONZERO_WEIGHTING * nonzero_loss


def kernel(reconstructed_image, target_image):
    total_n = float(reconstructed_image.size)
    rec = reconstructed_image.reshape(_PLANES, _H, _W)
    tgt = target_image.reshape(_PLANES, _H, _W)

    grid = _PLANES // _BLOCK_PLANES
    out = pl.pallas_call(
        functools.partial(_loss_body, total_n=total_n),
        grid=(grid,),
        in_specs=[
            pl.BlockSpec((_BLOCK_PLANES, _H, _W), lambda i: (i, 0, 0)),
            pl.BlockSpec((_BLOCK_PLANES, _H, _W), lambda i: (i, 0, 0)),
        ],
        out_specs=pl.BlockSpec(memory_space=pltpu.SMEM),
        out_shape=jax.ShapeDtypeStruct((1,), jnp.float32),
        scratch_shapes=[pltpu.SMEM((3,), jnp.float32)],
    )(rec, tgt)
    return out[0]


# speedup vs baseline: 1.3923x; 1.1487x over previous
import functools

import jax
import jax.numpy as jnp
from jax.experimental import pallas as pl
from jax.experimental.pallas import tpu as pltpu

_ZERO_WEIGHTING = 0.5
_NONZERO_WEIGHTING = 1.0

_PLANES = 384
_H = 224
_W = 224
_BLOCK_PLANES = 32


def _loss_body(rec_ref, tgt_ref, out_ref, acc_ref, *, total_n):
    i = pl.program_id(0)
    n = pl.num_programs(0)

    t = tgt_ref[...]
    r = rec_ref[...]
    zero = t == 0.0
    d = r - t
    ssq = jnp.sum(jnp.where(zero, d * d, 0.0))
    sab = jnp.sum(jnp.where(zero, 0.0, jnp.abs(d)))
    nz = jnp.sum(zero.astype(jnp.float32))

    @pl.when(i == 0)
    def _init():
        acc_ref[0] = 0.0
        acc_ref[1] = 0.0
        acc_ref[2] = 0.0

    acc_ref[0] += ssq
    acc_ref[1] += sab
    acc_ref[2] += nz

    @pl.when(i == n - 1)
    def _finish():
        n_zero = acc_ref[2]
        n_nonzero = total_n - n_zero
        zero_loss = jnp.where(n_zero > 0, acc_ref[0] / jnp.maximum(n_zero, 1.0), 0.0)
        nonzero_loss = jnp.where(
            n_nonzero > 0, acc_ref[1] / jnp.maximum(n_nonzero, 1.0), 0.0
        )
        out_ref[0] = _ZERO_WEIGHTING * zero_loss + _NONZERO_WEIGHTING * nonzero_loss


def kernel(reconstructed_image, target_image):
    total_n = float(reconstructed_image.size)
    rec = reconstructed_image.reshape(_PLANES, _H, _W)
    tgt = target_image.reshape(_PLANES, _H, _W)

    grid = _PLANES // _BLOCK_PLANES
    out = pl.pallas_call(
        functools.partial(_loss_body, total_n=total_n),
        grid=(grid,),
        in_specs=[
            pl.BlockSpec((_BLOCK_PLANES, _H, _W), lambda i: (i, 0, 0)),
            pl.BlockSpec((_BLOCK_PLANES, _H, _W), lambda i: (i, 0, 0)),
        ],
        out_specs=pl.BlockSpec(memory_space=pltpu.SMEM),
        out_shape=jax.ShapeDtypeStruct((1,), jnp.float32),
        scratch_shapes=[pltpu.SMEM((3,), jnp.float32)],
    )(rec, tgt)
    return out[0]


# speedup vs baseline: 1.5108x; 1.0851x over previous
import functools

import jax
import jax.numpy as jnp
from jax.experimental import pallas as pl
from jax.experimental.pallas import tpu as pltpu

_ZERO_WEIGHTING = 0.5
_NONZERO_WEIGHTING = 1.0

_PLANES = 384
_H = 224
_W = 224
_BLOCK_PLANES = 48
_SUB_PLANES = 12


def _loss_body(rec_ref, tgt_ref, out_ref, acc_ref, *, total_n):
    i = pl.program_id(0)
    n = pl.num_programs(0)

    ssq = jnp.float32(0.0)
    sab = jnp.float32(0.0)
    nz = jnp.float32(0.0)
    for c in range(0, _BLOCK_PLANES, _SUB_PLANES):
        t = tgt_ref[pl.ds(c, _SUB_PLANES)]
        r = rec_ref[pl.ds(c, _SUB_PLANES)]
        zero = t == 0.0
        d = r - t
        ssq += jnp.sum(jnp.where(zero, d * d, 0.0))
        sab += jnp.sum(jnp.where(zero, 0.0, jnp.abs(d)))
        nz += jnp.sum(zero.astype(jnp.float32))

    @pl.when(i == 0)
    def _init():
        acc_ref[0] = 0.0
        acc_ref[1] = 0.0
        acc_ref[2] = 0.0

    acc_ref[0] += ssq
    acc_ref[1] += sab
    acc_ref[2] += nz

    @pl.when(i == n - 1)
    def _finish():
        n_zero = acc_ref[2]
        n_nonzero = total_n - n_zero
        zero_loss = jnp.where(n_zero > 0, acc_ref[0] / jnp.maximum(n_zero, 1.0), 0.0)
        nonzero_loss = jnp.where(
            n_nonzero > 0, acc_ref[1] / jnp.maximum(n_nonzero, 1.0), 0.0
        )
        out_ref[0] = _ZERO_WEIGHTING * zero_loss + _NONZERO_WEIGHTING * nonzero_loss


def kernel(reconstructed_image, target_image):
    total_n = float(reconstructed_image.size)
    rec = reconstructed_image.reshape(_PLANES, _H, _W)
    tgt = target_image.reshape(_PLANES, _H, _W)

    grid = _PLANES // _BLOCK_PLANES
    out = pl.pallas_call(
        functools.partial(_loss_body, total_n=total_n),
        grid=(grid,),
        in_specs=[
            pl.BlockSpec((_BLOCK_PLANES, _H, _W), lambda i: (i, 0, 0)),
            pl.BlockSpec((_BLOCK_PLANES, _H, _W), lambda i: (i, 0, 0)),
        ],
        out_specs=pl.BlockSpec(memory_space=pltpu.SMEM),
        out_shape=jax.ShapeDtypeStruct((1,), jnp.float32),
        scratch_shapes=[pltpu.SMEM((3,), jnp.float32)],
    )(rec, tgt)
    return out[0]
